# baseline (speedup 1.0000x reference)
import math

import jax
import jax.numpy as jnp
from jax.experimental import pallas as pl
from jax.experimental.pallas import tpu as pltpu

_N = 2048
_C = 4
_BP = 256
_NP = _N // _BP
_PER = 0.001
_SIG1 = 1.0 / (1.0 + math.exp(-1.0))
_LOG2E = math.log2(math.e)


def _panel_chain(c_val, srow, a_sc, degs_all, odq_all, qoh_all, y_all):
    apq = a_sc[pl.ds(srow * _BP, _BP), :].astype(jnp.float32)
    dp = degs_all[pl.ds(srow * _BP, _BP), :]
    odq = odq_all[...]

    cnt_sub = dp - c_val - apq * odq
    ind_sub = jnp.minimum(cnt_sub, 1.0)
    ind_int = jnp.minimum(c_val, 1.0)
    numx = (-_SIG1 * _LOG2E) * cnt_sub + (-_LOG2E)
    den = _SIG1 * c_val + 1.0
    t2 = jnp.exp2(numx / den)
    v = t2 / (1.0 + t2)

    m1 = jnp.dot(v, y_all[...], preferred_element_type=jnp.float32)
    s1 = jnp.dot(ind_sub, qoh_all[...], preferred_element_type=jnp.float32)
    i1 = jnp.dot(ind_int, qoh_all[...], preferred_element_type=jnp.float32)
    return jnp.concatenate([m1, s1, i1], axis=1)


def _eloss_kernel(a0, a1, a2, a3, a4, a5, a6, a7, preds_ref, lab_ref, msk_ref,
                  out_ref,
                  a_sc, c_a, c_b, degs_all, odq_all, qoh_all, y_all, xp8_all,
                  acc24, nvec_acc, ce_acc):
    k = pl.program_id(0)
    blocks = (a0, a1, a2, a3, a4, a5, a6, a7)

    @pl.when(k == 0)
    def _precompute():
        acc24[...] = jnp.zeros_like(acc24)

        for b in range(_NP):
            a_sc[b * _BP:(b + 1) * _BP, :] = blocks[b][...].astype(jnp.bfloat16)

        ones_col = jnp.ones((_N, 1), dtype=jnp.bfloat16)
        degs_all[...] = jax.lax.dot_general(
            a_sc[...], ones_col, (((1,), (0,)), ((), ())),
            preferred_element_type=jnp.float32)

        for b in range(_NP):
            blk = a_sc[b * _BP:(b + 1) * _BP,
                       b * _BP:(b + 1) * _BP].astype(jnp.float32)
            ir = jax.lax.broadcasted_iota(jnp.int32, (_BP, _BP), 0)
            ic = jax.lax.broadcasted_iota(jnp.int32, (_BP, _BP), 1)
            diag = jnp.sum(blk * (ir == ic).astype(jnp.float32),
                           axis=0, keepdims=True)
            odq_all[:, b * _BP:(b + 1) * _BP] = 1.0 - diag

        preds = preds_ref[...]
        cls = jax.lax.broadcasted_iota(jnp.int32, (_N, _C), 1)
        oh = (lab_ref[...] == cls).astype(jnp.float32)
        qoh = oh * msk_ref[...]
        qoh_all[...] = qoh
        e_q = jnp.exp(preds)
        y_all[...] = jnp.concatenate(
            [e_q[:, i:i + 1] * qoh for i in range(_C)], axis=1)
        xp8_all[...] = jnp.concatenate(
            [qoh * jnp.exp(-preds), qoh], axis=1)

        nvec_acc[...] = jnp.sum(qoh, axis=0, keepdims=True)
        m = jnp.max(preds, axis=1, keepdims=True)
        lse = m + jnp.log(jnp.sum(jnp.exp(preds - m), axis=1, keepdims=True))
        pick = jnp.sum(oh * preds, axis=1, keepdims=True)
        ce_acc[...] = jnp.sum(lse - pick).reshape(1, 1)

    s_a = jnp.maximum(2 * k - 2, 0)
    s_b = jnp.maximum(2 * k - 1, 0)
    rhs_a = _panel_chain(c_a[...], s_a, a_sc, degs_all, odq_all,
                         qoh_all, y_all)
    rhs_b = _panel_chain(c_b[...], s_b, a_sc, degs_all, odq_all,
                         qoh_all, y_all)

    r_a = jnp.minimum(2 * k, _NP - 1)
    r_b = jnp.minimum(2 * k + 1, _NP - 1)
    c_a[...] = jax.lax.dot_general(
        a_sc[pl.ds(r_a * _BP, _BP), :], a_sc[...],
        (((1,), (0,)), ((), ())),
        preferred_element_type=jnp.float32)
    c_b[...] = jax.lax.dot_general(
        a_sc[pl.ds(r_b * _BP, _BP), :], a_sc[...],
        (((1,), (0,)), ((), ())),
        preferred_element_type=jnp.float32)

    @pl.when(k > 0)
    def _accumulate():
        lhs_a = xp8_all[pl.ds(s_a * _BP, _BP), :]
        lhs_b = xp8_all[pl.ds(s_b * _BP, _BP), :]
        acc24[...] += (
            jax.lax.dot_general(lhs_a, rhs_a, (((0,), (0,)), ((), ())),
                                preferred_element_type=jnp.float32)
            + jax.lax.dot_general(lhs_b, rhs_b, (((0,), (0,)), ((), ())),
                                  preferred_element_type=jnp.float32))

    @pl.when(k == _NP // 2)
    def _final():
        nv = nvec_acc[...]
        denom = jax.lax.dot_general(
            nv, nv, (((0,), (0,)), ((), ())),
            preferred_element_type=jnp.float32)
        inv = 1.0 / jnp.where(denom > 0.0, denom, 1.0)
        cond = jnp.logical_and(acc24[4:8, 16:20] > 0.0,
                               acc24[4:8, 20:24] > 0.0)
        pair = jnp.concatenate(
            [acc24[i:i + 1, 4 * i:4 * i + 4] for i in range(_C)], axis=0)
        i4r = jax.lax.broadcasted_iota(jnp.int32, (_C, _C), 0)
        i4c = jax.lax.broadcasted_iota(jnp.int32, (_C, _C), 1)
        offdiag = i4r != i4c
        contrib = jnp.where(jnp.logical_and(cond, offdiag), pair * inv, 0.0)
        out_ref[...] = ce_acc[...] / float(_N) + _PER * jnp.sum(contrib)


def kernel(preds, labels, mask, adj_matrix):
    a_i8 = adj_matrix.view(jnp.int8)
    lab2 = labels.reshape(_N, 1).astype(jnp.int32)
    msk2 = mask.reshape(_N, 1).astype(jnp.float32)

    blk_specs = [
        pl.BlockSpec((_BP, _N), (lambda k, b=b: (b, 0))) for b in range(_NP)
    ]
    out = pl.pallas_call(
        _eloss_kernel,
        grid=(_NP // 2 + 1,),
        in_specs=blk_specs + [
            pl.BlockSpec((_N, _C), lambda k: (0, 0)),
            pl.BlockSpec((_N, 1), lambda k: (0, 0)),
            pl.BlockSpec((_N, 1), lambda k: (0, 0)),
        ],
        out_specs=pl.BlockSpec((1, 1), lambda k: (0, 0)),
        out_shape=jax.ShapeDtypeStruct((1, 1), jnp.float32),
        scratch_shapes=[
            pltpu.VMEM((_N, _N), jnp.bfloat16),
            pltpu.VMEM((_BP, _N), jnp.float32),
            pltpu.VMEM((_BP, _N), jnp.float32),
            pltpu.VMEM((_N, 1), jnp.float32),
            pltpu.VMEM((1, _N), jnp.float32),
            pltpu.VMEM((_N, _C), jnp.float32),
            pltpu.VMEM((_N, 4 * _C), jnp.float32),
            pltpu.VMEM((_N, 2 * _C), jnp.float32),
            pltpu.VMEM((2 * _C, 6 * _C), jnp.float32),
            pltpu.VMEM((1, _C), jnp.float32),
            pltpu.VMEM((1, 1), jnp.float32),
        ],
    )(*([a_i8] * _NP), preds, lab2, msk2)
    return out.reshape(())


# speedup vs baseline: 1.2133x; 1.2133x over previous
import math

import jax
import jax.numpy as jnp
from jax.experimental import pallas as pl
from jax.experimental.pallas import tpu as pltpu

_N = 2048
_C = 4
_BP = 256
_NP = _N // _BP
_PER = 0.001
_SIG1 = 1.0 / (1.0 + math.exp(-1.0))
_LOG2E = math.log2(math.e)


def _panel_chain(c_val, srow, a_sc, degs_all, odq_all, qoh_all, y_all):
    apq = a_sc[pl.ds(srow * _BP, _BP), :].astype(jnp.float32)
    dp = degs_all[pl.ds(srow * _BP, _BP), :]
    odq = odq_all[...]

    cnt_sub = dp - c_val - apq * odq
    ind_sub = jnp.minimum(cnt_sub, 1.0)
    ind_int = jnp.minimum(c_val, 1.0)
    numx = (-_SIG1 * _LOG2E) * cnt_sub + (-_LOG2E)
    den = _SIG1 * c_val + 1.0
    t2 = jnp.exp2(numx / den)
    v = t2 / (1.0 + t2)

    m1 = jnp.dot(v, y_all[...], preferred_element_type=jnp.float32)
    s1 = jnp.dot(ind_sub, qoh_all[...], preferred_element_type=jnp.float32)
    i1 = jnp.dot(ind_int, qoh_all[...], preferred_element_type=jnp.float32)
    return jnp.concatenate([m1, s1, i1], axis=1)


def _eloss_kernel(a_sc, preds_ref, lab_ref, msk_ref,
                  out_ref,
                  c_a, c_b, degs_all, odq_all, qoh_all, y_all, xp8_all,
                  acc24, nvec_acc, ce_acc):
    k = pl.program_id(0)

    @pl.when(k == 0)
    def _precompute():
        acc24[...] = jnp.zeros_like(acc24)

        ones_col = jnp.ones((_N, 1), dtype=jnp.float8_e4m3fn)
        degs_all[...] = jax.lax.dot_general(
            a_sc[...], ones_col, (((1,), (0,)), ((), ())),
            preferred_element_type=jnp.float32)

        for b in range(_NP):
            blk = a_sc[b * _BP:(b + 1) * _BP,
                       b * _BP:(b + 1) * _BP].astype(jnp.float32)
            ir = jax.lax.broadcasted_iota(jnp.int32, (_BP, _BP), 0)
            ic = jax.lax.broadcasted_iota(jnp.int32, (_BP, _BP), 1)
            diag = jnp.sum(blk * (ir == ic).astype(jnp.float32),
                           axis=0, keepdims=True)
            odq_all[:, b * _BP:(b + 1) * _BP] = 1.0 - diag

        preds = preds_ref[...]
        cls = jax.lax.broadcasted_iota(jnp.int32, (_N, _C), 1)
        oh = (lab_ref[...] == cls).astype(jnp.float32)
        qoh = oh * msk_ref[...]
        qoh_all[...] = qoh
        e_q = jnp.exp(preds)
        y_all[...] = jnp.concatenate(
            [e_q[:, i:i + 1] * qoh for i in range(_C)], axis=1)
        xp8_all[...] = jnp.concatenate(
            [qoh * jnp.exp(-preds), qoh], axis=1)

        nvec_acc[...] = jnp.sum(qoh, axis=0, keepdims=True)
        m = jnp.max(preds, axis=1, keepdims=True)
        lse = m + jnp.log(jnp.sum(jnp.exp(preds - m), axis=1, keepdims=True))
        pick = jnp.sum(oh * preds, axis=1, keepdims=True)
        ce_acc[...] = jnp.sum(lse - pick).reshape(1, 1)

    s_a = jnp.maximum(2 * k - 2, 0)
    s_b = jnp.maximum(2 * k - 1, 0)
    rhs_a = _panel_chain(c_a[...], s_a, a_sc, degs_all, odq_all,
                         qoh_all, y_all)
    rhs_b = _panel_chain(c_b[...], s_b, a_sc, degs_all, odq_all,
                         qoh_all, y_all)

    r_a = jnp.minimum(2 * k, _NP - 1)
    r_b = jnp.minimum(2 * k + 1, _NP - 1)
    c_a[...] = jax.lax.dot_general(
        a_sc[pl.ds(r_a * _BP, _BP), :], a_sc[...],
        (((1,), (0,)), ((), ())),
        preferred_element_type=jnp.float32)
    c_b[...] = jax.lax.dot_general(
        a_sc[pl.ds(r_b * _BP, _BP), :], a_sc[...],
        (((1,), (0,)), ((), ())),
        preferred_element_type=jnp.float32)

    @pl.when(k > 0)
    def _accumulate():
        lhs_a = xp8_all[pl.ds(s_a * _BP, _BP), :]
        lhs_b = xp8_all[pl.ds(s_b * _BP, _BP), :]
        acc24[...] += (
            jax.lax.dot_general(lhs_a, rhs_a, (((0,), (0,)), ((), ())),
                                preferred_element_type=jnp.float32)
            + jax.lax.dot_general(lhs_b, rhs_b, (((0,), (0,)), ((), ())),
                                  preferred_element_type=jnp.float32))

    @pl.when(k == _NP // 2)
    def _final():
        nv = nvec_acc[...]
        denom = jax.lax.dot_general(
            nv, nv, (((0,), (0,)), ((), ())),
            preferred_element_type=jnp.float32)
        inv = 1.0 / jnp.where(denom > 0.0, denom, 1.0)
        cond = jnp.logical_and(acc24[4:8, 16:20] > 0.0,
                               acc24[4:8, 20:24] > 0.0)
        pair = jnp.concatenate(
            [acc24[i:i + 1, 4 * i:4 * i + 4] for i in range(_C)], axis=0)
        i4r = jax.lax.broadcasted_iota(jnp.int32, (_C, _C), 0)
        i4c = jax.lax.broadcasted_iota(jnp.int32, (_C, _C), 1)
        offdiag = i4r != i4c
        contrib = jnp.where(jnp.logical_and(cond, offdiag), pair * inv, 0.0)
        out_ref[...] = ce_acc[...] / float(_N) + _PER * jnp.sum(contrib)


def kernel(preds, labels, mask, adj_matrix):
    a_f8 = adj_matrix.astype(jnp.float8_e4m3fn)
    lab2 = labels.reshape(_N, 1).astype(jnp.int32)
    msk2 = mask.reshape(_N, 1).astype(jnp.float32)

    out = pl.pallas_call(
        _eloss_kernel,
        grid=(_NP // 2 + 1,),
        in_specs=[
            pl.BlockSpec((_N, _N), lambda k: (0, 0)),
            pl.BlockSpec((_N, _C), lambda k: (0, 0)),
            pl.BlockSpec((_N, 1), lambda k: (0, 0)),
            pl.BlockSpec((_N, 1), lambda k: (0, 0)),
        ],
        out_specs=pl.BlockSpec((1, 1), lambda k: (0, 0)),
        out_shape=jax.ShapeDtypeStruct((1, 1), jnp.float32),
        scratch_shapes=[
            pltpu.VMEM((_BP, _N), jnp.float32),
            pltpu.VMEM((_BP, _N), jnp.float32),
            pltpu.VMEM((_N, 1), jnp.float32),
            pltpu.VMEM((1, _N), jnp.float32),
            pltpu.VMEM((_N, _C), jnp.float32),
            pltpu.VMEM((_N, 4 * _C), jnp.float32),
            pltpu.VMEM((_N, 2 * _C), jnp.float32),
            pltpu.VMEM((2 * _C, 6 * _C), jnp.float32),
            pltpu.VMEM((1, _C), jnp.float32),
            pltpu.VMEM((1, 1), jnp.float32),
        ],
    )(a_f8, preds, lab2, msk2)
    return out.reshape(())


# speedup vs baseline: 1.2180x; 1.0038x over previous
import math

import jax
import jax.numpy as jnp
from jax.experimental import pallas as pl
from jax.experimental.pallas import tpu as pltpu

_N = 2048
_C = 4
_BP = 256
_NP = _N // _BP
_PER = 0.001
_SIG1 = 1.0 / (1.0 + math.exp(-1.0))
_LOG2E = math.log2(math.e)


def _panel_chain(c_val, srow, a_sc, degs_all, odq_all, qoh_all, y_all):
    apq = a_sc[pl.ds(srow * _BP, _BP), :].astype(jnp.float32)
    dp = degs_all[pl.ds(srow * _BP, _BP), :]
    odq = odq_all[...]

    cnt_sub = dp - c_val - apq * odq
    ind_sub = jnp.minimum(cnt_sub, 1.0)
    ind_int = jnp.minimum(c_val, 1.0)
    numx = (-_SIG1 * _LOG2E) * cnt_sub + (-_LOG2E)
    den = _SIG1 * c_val + 1.0
    t2 = jnp.exp2(numx / den)
    v = t2 / (1.0 + t2)

    m1 = jnp.dot(v, y_all[...], preferred_element_type=jnp.float32)
    s1 = jnp.dot(ind_sub, qoh_all[...], preferred_element_type=jnp.float32)
    i1 = jnp.dot(ind_int, qoh_all[...], preferred_element_type=jnp.float32)
    return jnp.concatenate([m1, s1, i1], axis=1)


def _eloss_kernel(a_in0, a_in1, a_in2, a_in3, preds_ref, lab_ref, msk_ref,
                  out_ref,
                  a_sc, c_a, c_b, degs_all, odq_all, qoh_all, y_all, xp8_all,
                  acc24, nvec_acc, ce_acc):
    k = pl.program_id(0)
    quarters = (a_in0, a_in1, a_in2, a_in3)

    @pl.when(k == 0)
    def _precompute():
        acc24[...] = jnp.zeros_like(acc24)

        for b in range(4):
            a_sc[b * (_N // 4):(b + 1) * (_N // 4), :] = quarters[b][...]

        ones_col = jnp.ones((_N, 1), dtype=jnp.float8_e4m3fn)
        degs_all[...] = jax.lax.dot_general(
            a_sc[...], ones_col, (((1,), (0,)), ((), ())),
            preferred_element_type=jnp.float32)

        for b in range(_NP):
            blk = a_sc[b * _BP:(b + 1) * _BP,
                       b * _BP:(b + 1) * _BP].astype(jnp.float32)
            ir = jax.lax.broadcasted_iota(jnp.int32, (_BP, _BP), 0)
            ic = jax.lax.broadcasted_iota(jnp.int32, (_BP, _BP), 1)
            diag = jnp.sum(blk * (ir == ic).astype(jnp.float32),
                           axis=0, keepdims=True)
            odq_all[:, b * _BP:(b + 1) * _BP] = 1.0 - diag

        preds = preds_ref[...]
        cls = jax.lax.broadcasted_iota(jnp.int32, (_N, _C), 1)
        oh = (lab_ref[...] == cls).astype(jnp.float32)
        qoh = oh * msk_ref[...]
        qoh_all[...] = qoh
        e_q = jnp.exp(preds)
        y_all[...] = jnp.concatenate(
            [e_q[:, i:i + 1] * qoh for i in range(_C)], axis=1)
        xp8_all[...] = jnp.concatenate(
            [qoh * jnp.exp(-preds), qoh], axis=1)

        nvec_acc[...] = jnp.sum(qoh, axis=0, keepdims=True)
        m = jnp.max(preds, axis=1, keepdims=True)
        lse = m + jnp.log(jnp.sum(jnp.exp(preds - m), axis=1, keepdims=True))
        pick = jnp.sum(oh * preds, axis=1, keepdims=True)
        ce_acc[...] = jnp.sum(lse - pick).reshape(1, 1)

    s_a = jnp.maximum(2 * k - 2, 0)
    s_b = jnp.maximum(2 * k - 1, 0)
    rhs_a = _panel_chain(c_a[...], s_a, a_sc, degs_all, odq_all,
                         qoh_all, y_all)
    rhs_b = _panel_chain(c_b[...], s_b, a_sc, degs_all, odq_all,
                         qoh_all, y_all)

    r_a = jnp.minimum(2 * k, _NP - 1)
    r_b = jnp.minimum(2 * k + 1, _NP - 1)
    c_a[...] = jax.lax.dot_general(
        a_sc[pl.ds(r_a * _BP, _BP), :], a_sc[...],
        (((1,), (0,)), ((), ())),
        preferred_element_type=jnp.float32)
    c_b[...] = jax.lax.dot_general(
        a_sc[pl.ds(r_b * _BP, _BP), :], a_sc[...],
        (((1,), (0,)), ((), ())),
        preferred_element_type=jnp.float32)

    @pl.when(k > 0)
    def _accumulate():
        lhs_a = xp8_all[pl.ds(s_a * _BP, _BP), :]
        lhs_b = xp8_all[pl.ds(s_b * _BP, _BP), :]
        acc24[...] += (
            jax.lax.dot_general(lhs_a, rhs_a, (((0,), (0,)), ((), ())),
                                preferred_element_type=jnp.float32)
            + jax.lax.dot_general(lhs_b, rhs_b, (((0,), (0,)), ((), ())),
                                  preferred_element_type=jnp.float32))

    @pl.when(k == _NP // 2)
    def _final():
        nv = nvec_acc[...]
        denom = jax.lax.dot_general(
            nv, nv, (((0,), (0,)), ((), ())),
            preferred_element_type=jnp.float32)
        inv = 1.0 / jnp.where(denom > 0.0, denom, 1.0)
        cond = jnp.logical_and(acc24[4:8, 16:20] > 0.0,
                               acc24[4:8, 20:24] > 0.0)
        pair = jnp.concatenate(
            [acc24[i:i + 1, 4 * i:4 * i + 4] for i in range(_C)], axis=0)
        i4r = jax.lax.broadcasted_iota(jnp.int32, (_C, _C), 0)
        i4c = jax.lax.broadcasted_iota(jnp.int32, (_C, _C), 1)
        offdiag = i4r != i4c
        contrib = jnp.where(jnp.logical_and(cond, offdiag), pair * inv, 0.0)
        out_ref[...] = ce_acc[...] / float(_N) + _PER * jnp.sum(contrib)


def kernel(preds, labels, mask, adj_matrix):
    a_f8 = adj_matrix.astype(jnp.float8_e4m3fn)
    lab2 = labels.reshape(_N, 1).astype(jnp.int32)
    msk2 = mask.reshape(_N, 1).astype(jnp.float32)

    out = pl.pallas_call(
        _eloss_kernel,
        grid=(_NP // 2 + 1,),
        in_specs=[
            pl.BlockSpec((_N // 4, _N), lambda k: (0, 0)),
            pl.BlockSpec((_N // 4, _N), lambda k: (1, 0)),
            pl.BlockSpec((_N // 4, _N), lambda k: (2, 0)),
            pl.BlockSpec((_N // 4, _N), lambda k: (3, 0)),
            pl.BlockSpec((_N, _C), lambda k: (0, 0)),
            pl.BlockSpec((_N, 1), lambda k: (0, 0)),
            pl.BlockSpec((_N, 1), lambda k: (0, 0)),
        ],
        out_specs=pl.BlockSpec((1, 1), lambda k: (0, 0)),
        out_shape=jax.ShapeDtypeStruct((1, 1), jnp.float32),
        scratch_shapes=[
            pltpu.VMEM((_N, _N), jnp.float8_e4m3fn),
            pltpu.VMEM((_BP, _N), jnp.float32),
            pltpu.VMEM((_BP, _N), jnp.float32),
            pltpu.VMEM((_N, 1), jnp.float32),
            pltpu.VMEM((1, _N), jnp.float32),
            pltpu.VMEM((_N, _C), jnp.float32),
            pltpu.VMEM((_N, 4 * _C), jnp.float32),
            pltpu.VMEM((_N, 2 * _C), jnp.float32),
            pltpu.VMEM((2 * _C, 6 * _C), jnp.float32),
            pltpu.VMEM((1, _C), jnp.float32),
            pltpu.VMEM((1, 1), jnp.float32),
        ],
    )(a_f8, a_f8, a_f8, a_f8, preds, lab2, msk2)
    return out.reshape(())
